# baseline (speedup 1.0000x reference)
import functools

import jax
import jax.numpy as jnp
from jax import lax
from jax.experimental import pallas as pl
from jax.experimental.pallas import tpu as pltpu
from jax.experimental.pallas import tpu_sc as plsc

GRID = 129
NC = 2
NS = 16
NW = NC * NS
CH = 8192
LANES = 16


def _body(asu_hbm, h_hbm, k_hbm, l_hbm, tab_hbm, out_hbm,
          asu_v, h_v, k_v, l_v, idx_v, o_v, sem):
    wid = lax.axis_index("s") * NC + lax.axis_index("c")
    b_per_w = asu_hbm.shape[0] // NW
    n_chunks = b_per_w // CH
    base = wid * b_per_w

    def chunk_body(c, carry):
        off = base + c * CH
        pltpu.sync_copy(asu_hbm.at[pl.ds(off, CH)], asu_v)
        pltpu.sync_copy(h_hbm.at[pl.ds(off, CH)], h_v)
        pltpu.sync_copy(k_hbm.at[pl.ds(off, CH)], k_v)
        pltpu.sync_copy(l_hbm.at[pl.ds(off, CH)], l_v)

        def vec_body(i, carry2):
            s = pl.ds(i * LANES, LANES)
            idx_v[s] = ((asu_v[s] * GRID + h_v[s]) * GRID + k_v[s]) * GRID + l_v[s]
            return carry2

        lax.fori_loop(0, CH // LANES, vec_body, 0, unroll=4)
        pltpu.async_copy(tab_hbm.at[idx_v], o_v, sem).wait()
        pltpu.sync_copy(o_v, out_hbm.at[pl.ds(off, CH)])
        return carry

    lax.fori_loop(0, n_chunks, chunk_body, 0)


def kernel(asu_id, hkl, miller_id):
    B = asu_id.shape[0]
    asu32 = asu_id.astype(jnp.int32)
    hkl32 = hkl.astype(jnp.int32)
    h, k, l = hkl32[:, 0], hkl32[:, 1], hkl32[:, 2]
    tab = miller_id.reshape(-1)

    mesh = plsc.VectorSubcoreMesh(core_axis_name="c", subcore_axis_name="s")
    run = functools.partial(
        pl.kernel,
        mesh=mesh,
        out_type=jax.ShapeDtypeStruct((B,), jnp.int32),
        scratch_types=[
            pltpu.VMEM((CH,), jnp.int32),
            pltpu.VMEM((CH,), jnp.int32),
            pltpu.VMEM((CH,), jnp.int32),
            pltpu.VMEM((CH,), jnp.int32),
            pltpu.VMEM((CH,), jnp.int32),
            pltpu.VMEM((CH,), jnp.int32),
            pltpu.SemaphoreType.DMA,
        ],
    )(_body)
    return run(asu32, h, k, l, tab)
